# baseline (speedup 1.0000x reference)
import jax
import jax.numpy as jnp
from functools import partial
from jax.experimental import pallas as pl
from jax.experimental.pallas import tpu as pltpu


def _fused_kernel(x_ref, adj_ref, xgb_ref, wbd_ref, wbil_ref, bias_ref,
                  a_ref, b_ref, out_ref, prevc_scr, v0_scr,
                  *, BS, S, F, H, B, NB):
    i = pl.program_id(0)

    x2 = x_ref[...]
    adj2 = adj_ref[...]
    a = a_ref[...]
    b = b_ref[...]

    mixed = []
    for ii in range(S):
        acc = x2[:, 0:F] * adj2[:, ii * S:ii * S + 1]
        for j in range(1, S):
            acc = acc + x2[:, j * F:(j + 1) * F] * adj2[:, ii * S + j:ii * S + j + 1]
        mixed.append(acc)
    mcat = jnp.concatenate(mixed, axis=1)

    o = jnp.dot(mcat, wbd_ref[...], preferred_element_type=jnp.float32)
    o = o + bias_ref[...]
    t = jnp.where(o >= 0, o, a * o)
    h = jnp.maximum(t, 0.0)

    c = h[:, 0:H]
    for j in range(1, S - 1):
        c = c + h[:, j * H:(j + 1) * H]
    c = c * (1.0 / (S - 1))
    hmv = 0.5 * h[:, (S - 1) * H:S * H] + xgb_ref[...]
    v = jnp.dot(hmv, wbil_ref[...], preferred_element_type=jnp.float32)

    prev_c = prevc_scr[...]
    prevc_scr[...] = c[BS - 1:, :]

    @pl.when(i == 0)
    def _():
        v0_scr[...] = v[0:1, :]

    cr = jnp.concatenate([prev_c, c[:BS - 1, :]], axis=0)
    s1 = jnp.sum(v * c, axis=1, keepdims=True)
    s2 = jnp.sum(v * cr, axis=1, keepdims=True)
    out_ref[pl.ds(i * BS, BS), :] = s1 + b
    out_ref[pl.ds(B + i * BS, BS), :] = s2 + b

    @pl.when(i == NB - 1)
    def _():
        patch = jnp.sum(v0_scr[...] * c[BS - 1:, :], axis=1, keepdims=True)
        out_ref[B:B + 1, :] = patch + b


@jax.jit
def kernel(x, adj, x_g_b, W_fc, bias_gc, prelu_a, W_bil, b_bil):
    B, S, F = x.shape
    H = W_fc.shape[0]
    BS = 512
    NB = B // BS

    eye = jnp.eye(S, dtype=W_fc.dtype)
    wbd = jnp.einsum('pq,hf->pfqh', eye, W_fc).reshape(S * F, S * H)
    wbil = W_bil.reshape(H, H)
    biascat = jnp.tile(bias_gc, (S,)).reshape(1, S * H)
    a2 = jnp.reshape(prelu_a, (1, 1)).astype(jnp.float32)
    b2 = jnp.reshape(b_bil, (1, 1)).astype(jnp.float32)

    body = partial(_fused_kernel, BS=BS, S=S, F=F, H=H, B=B, NB=NB)
    out = pl.pallas_call(
        body,
        grid=(NB,),
        in_specs=[
            pl.BlockSpec((BS, S * F), lambda i: (i, 0)),
            pl.BlockSpec((BS, S * S), lambda i: (i, 0)),
            pl.BlockSpec((BS, H), lambda i: (i, 0)),
            pl.BlockSpec((S * F, S * H), lambda i: (0, 0)),
            pl.BlockSpec((H, H), lambda i: (0, 0)),
            pl.BlockSpec((1, S * H), lambda i: (0, 0)),
            pl.BlockSpec((1, 1), lambda i: (0, 0)),
            pl.BlockSpec((1, 1), lambda i: (0, 0)),
        ],
        out_specs=pl.BlockSpec((2 * B, 1), lambda i: (0, 0)),
        out_shape=jax.ShapeDtypeStruct((2 * B, 1), jnp.float32),
        scratch_shapes=[pltpu.VMEM((1, H), jnp.float32),
                        pltpu.VMEM((1, H), jnp.float32)],
    )(x.reshape(B, S * F), adj.reshape(B, S * S), x_g_b, wbd, wbil, biascat, a2, b2)

    return out


# speedup vs baseline: 1.3682x; 1.3682x over previous
import jax
import jax.numpy as jnp
from functools import partial
from jax.experimental import pallas as pl
from jax.experimental.pallas import tpu as pltpu


def _fused_kernel(x_ref, adj_ref, xgb_ref, wbd_ref, wbil_ref, bias_ref,
                  a_ref, b_ref, out_ref, prevc_scr, v0_scr,
                  *, BS, S, F, H, B, NB):
    i = pl.program_id(0)

    x2 = x_ref[...].reshape(BS, S * F)
    adj2 = adj_ref[...]
    a = a_ref[...]
    b = b_ref[...]

    mixed = []
    for ii in range(S):
        acc = x2[:, 0:F] * adj2[:, ii * S:ii * S + 1]
        for j in range(1, S):
            acc = acc + x2[:, j * F:(j + 1) * F] * adj2[:, ii * S + j:ii * S + j + 1]
        mixed.append(acc)
    mcat = jnp.concatenate(mixed, axis=1)

    o = jnp.dot(mcat, wbd_ref[...], preferred_element_type=jnp.float32)
    o = o + bias_ref[...]
    t = jnp.where(o >= 0, o, a * o)
    h = jnp.maximum(t, 0.0)

    c = h[:, 0:H]
    for j in range(1, S - 1):
        c = c + h[:, j * H:(j + 1) * H]
    c = c * (1.0 / (S - 1))
    hmv = 0.5 * h[:, (S - 1) * H:S * H] + xgb_ref[...]
    v = jnp.dot(hmv, wbil_ref[...], preferred_element_type=jnp.float32)

    prev_c = prevc_scr[...]
    prevc_scr[...] = c[BS - 1:, :]

    @pl.when(i == 0)
    def _():
        v0_scr[...] = v[0:1, :]

    cr = jnp.concatenate([prev_c, c[:BS - 1, :]], axis=0)
    s1 = jnp.sum(v * c, axis=1, keepdims=True)
    s2 = jnp.sum(v * cr, axis=1, keepdims=True)
    out_ref[pl.ds(i * BS, BS), :] = s1 + b
    out_ref[pl.ds(B + i * BS, BS), :] = s2 + b

    @pl.when(i == NB - 1)
    def _():
        patch = jnp.sum(v0_scr[...] * c[BS - 1:, :], axis=1, keepdims=True)
        out_ref[B:B + 1, :] = patch + b


@jax.jit
def kernel(x, adj, x_g_b, W_fc, bias_gc, prelu_a, W_bil, b_bil):
    B, S, F = x.shape
    H = W_fc.shape[0]
    BS = 512
    NB = B // BS

    eye = jnp.eye(S, dtype=W_fc.dtype)
    wbd = jnp.einsum('pq,hf->pfqh', eye, W_fc).reshape(S * F, S * H)
    wbil = W_bil.reshape(H, H)
    biascat = jnp.tile(bias_gc, (S,)).reshape(1, S * H)
    a2 = jnp.reshape(prelu_a, (1, 1)).astype(jnp.float32)
    b2 = jnp.reshape(b_bil, (1, 1)).astype(jnp.float32)

    body = partial(_fused_kernel, BS=BS, S=S, F=F, H=H, B=B, NB=NB)
    out = pl.pallas_call(
        body,
        grid=(NB,),
        in_specs=[
            pl.BlockSpec((BS, S, F), lambda i: (i, 0, 0)),
            pl.BlockSpec((BS, S * S), lambda i: (i, 0)),
            pl.BlockSpec((BS, H), lambda i: (i, 0)),
            pl.BlockSpec((S * F, S * H), lambda i: (0, 0)),
            pl.BlockSpec((H, H), lambda i: (0, 0)),
            pl.BlockSpec((1, S * H), lambda i: (0, 0)),
            pl.BlockSpec((1, 1), lambda i: (0, 0)),
            pl.BlockSpec((1, 1), lambda i: (0, 0)),
        ],
        out_specs=pl.BlockSpec((2 * B, 1), lambda i: (0, 0)),
        out_shape=jax.ShapeDtypeStruct((2 * B, 1), jnp.float32),
        scratch_shapes=[pltpu.VMEM((1, H), jnp.float32),
                        pltpu.VMEM((1, H), jnp.float32)],
    )(x, adj.reshape(B, S * S), x_g_b, wbd, wbil, biascat, a2, b2)

    return out
